# Initial kernel scaffold; baseline (speedup 1.0000x reference)
import functools

import jax
import jax.numpy as jnp
from jax import lax
from jax.experimental import pallas as pl
from jax.experimental.pallas import tpu as pltpu
from jax.experimental.pallas import tpu_sc as plsc

N_NODES = 10000
N_EDGES = 320000
D = 128

NC = 2
NS = 16
NW = NC * NS

CHUNK = 125
EDGES_PER_TILE = N_EDGES // NW
CHUNKS = EDGES_PER_TILE // CHUNK

ROWS_PER_TILE = N_NODES // NS
DEG_STRIDE = 640


def _sc_aggregate_body(x_hbm, src_hbm, dst_hbm, acc_out, deg_out,
                       src_v, dst_v, rows_v, ones_v, zrow_v,
                       acc_sh, deg_sh, gsem):
    c = lax.axis_index("c")
    s = lax.axis_index("s")
    wid = c * NS + s

    def zero_rows(k, carry):
        rows_v[k // 8, pl.ds((k % 8) * 16, 16)] = jnp.zeros((16,), jnp.float32)
        return carry
    lax.fori_loop(0, CHUNK * 8, zero_rows, 0)

    def zero_zrow(k, carry):
        zrow_v[pl.ds(k * 16, 16)] = jnp.zeros((16,), jnp.float32)
        return carry
    lax.fori_loop(0, DEG_STRIDE // 16, zero_zrow, 0)

    def fill_ones(k, carry):
        ones_v[pl.ds(k * 16, 16)] = jnp.ones((16,), jnp.float32)
        return carry
    lax.fori_loop(0, 8, fill_ones, 0)

    for t in range(ROWS_PER_TILE // CHUNK):
        pltpu.sync_copy(rows_v, acc_sh.at[pl.ds(s * ROWS_PER_TILE + t * CHUNK, CHUNK)])

    @pl.when(s < NS - 1)
    def _():
        pltpu.sync_copy(zrow_v, deg_sh.at[pl.ds(s * DEG_STRIDE, DEG_STRIDE)])

    @pl.when(s == NS - 1)
    def _():
        pltpu.sync_copy(zrow_v.at[pl.ds(0, N_NODES - 15 * DEG_STRIDE)],
                        deg_sh.at[pl.ds(15 * DEG_STRIDE, N_NODES - 15 * DEG_STRIDE)])

    pltpu.sync_copy(src_hbm.at[wid], src_v)
    pltpu.sync_copy(dst_hbm.at[wid], dst_v)

    plsc.subcore_barrier()

    def body(j, carry):
        pltpu.async_copy(x_hbm.at[src_v.at[j]], rows_v, gsem).wait()
        pltpu.sync_copy(rows_v, acc_sh.at[dst_v.at[j]], add=True)
        pltpu.sync_copy(ones_v.at[pl.ds(0, CHUNK)], deg_sh.at[dst_v.at[j]], add=True)
        return carry
    lax.fori_loop(0, CHUNKS, body, 0)

    plsc.subcore_barrier()

    pltpu.sync_copy(acc_sh.at[pl.ds(s * ROWS_PER_TILE, ROWS_PER_TILE)],
                    acc_out.at[c, pl.ds(s * ROWS_PER_TILE, ROWS_PER_TILE)])

    @pl.when(s < NS - 1)
    def _():
        pltpu.sync_copy(deg_sh.at[pl.ds(s * DEG_STRIDE, DEG_STRIDE)],
                        deg_out.at[c, pl.ds(s * DEG_STRIDE, DEG_STRIDE)])

    @pl.when(s == NS - 1)
    def _():
        pltpu.sync_copy(deg_sh.at[pl.ds(15 * DEG_STRIDE, N_NODES - 15 * DEG_STRIDE)],
                        deg_out.at[c, pl.ds(15 * DEG_STRIDE, N_NODES - 15 * DEG_STRIDE)])


_sc_aggregate = functools.partial(
    pl.kernel,
    mesh=plsc.VectorSubcoreMesh(core_axis_name="c", subcore_axis_name="s"),
    out_type=[
        jax.ShapeDtypeStruct((NC, N_NODES, D), jnp.float32),
        jax.ShapeDtypeStruct((NC, N_NODES), jnp.float32),
    ],
    scratch_types=[
        pltpu.VMEM((CHUNKS, CHUNK), jnp.int32),
        pltpu.VMEM((CHUNKS, CHUNK), jnp.int32),
        pltpu.VMEM((CHUNK, D), jnp.float32),
        pltpu.VMEM((D,), jnp.float32),
        pltpu.VMEM((DEG_STRIDE,), jnp.float32),
        pltpu.VMEM_SHARED((N_NODES, D), jnp.float32),
        pltpu.VMEM_SHARED((N_NODES,), jnp.float32),
        pltpu.SemaphoreType.DMA,
    ],
)(_sc_aggregate_body)


BM = 2000


def _finish_body(acc_ref, deg_ref, w_ref, b_ref, o_ref):
    a = acc_ref[0] + acc_ref[1]
    d = jnp.maximum(deg_ref[0] + deg_ref[1], 1.0)
    y = jnp.dot(a, w_ref[...], preferred_element_type=jnp.float32)
    o_ref[...] = y / d[:, None] + b_ref[...]


def _finish(acc, deg, W, b2):
    return pl.pallas_call(
        _finish_body,
        grid=(N_NODES // BM,),
        in_specs=[
            pl.BlockSpec((NC, BM, D), lambda i: (0, i, 0)),
            pl.BlockSpec((NC, BM), lambda i: (0, i)),
            pl.BlockSpec((D, D), lambda i: (0, 0)),
            pl.BlockSpec((1, D), lambda i: (0, 0)),
        ],
        out_specs=pl.BlockSpec((BM, D), lambda i: (i, 0)),
        out_shape=jax.ShapeDtypeStruct((N_NODES, D), jnp.float32),
    )(acc, deg, W, b2)


def kernel(x, edge_index, W, b):
    dst = edge_index[0].reshape(NW, CHUNKS, CHUNK)
    src = edge_index[1].reshape(NW, CHUNKS, CHUNK)
    acc, deg = _sc_aggregate(x, src, dst)
    return _finish(acc, deg, W, b.reshape(1, D))


# speedup vs baseline: 9.3712x; 9.3712x over previous
import functools

import jax
import jax.numpy as jnp
from jax import lax
from jax.experimental import pallas as pl
from jax.experimental.pallas import tpu as pltpu
from jax.experimental.pallas import tpu_sc as plsc

N_NODES = 10000
N_EDGES = 320000
D = 128

NC = 2
NS = 16
NW = NC * NS

CHUNK = 125
EDGES_PER_TILE = N_EDGES // NW
CHUNKS = EDGES_PER_TILE // CHUNK

ROWS_PER_TILE = N_NODES // NS
DEG_STRIDE = 640
N_PAD = NS * DEG_STRIDE


def _sc_aggregate_body(x_hbm, src_hbm, dst_hbm, acc_out, deg_out,
                       src_v, dst_v, rows_v, ones_v, zrow_v,
                       acc_sh, deg_sh, gsem):
    c = lax.axis_index("c")
    s = lax.axis_index("s")
    wid = c * NS + s

    def zero_rows(k, carry):
        rows_v[k // 8, pl.ds((k % 8) * 16, 16)] = jnp.zeros((16,), jnp.float32)
        return carry
    lax.fori_loop(0, CHUNK * 8, zero_rows, 0)

    def zero_zrow(k, carry):
        zrow_v[pl.ds(k * 16, 16)] = jnp.zeros((16,), jnp.float32)
        return carry
    lax.fori_loop(0, DEG_STRIDE // 16, zero_zrow, 0)

    def fill_ones(k, carry):
        ones_v[pl.ds(k * 16, 16)] = jnp.ones((16,), jnp.float32)
        return carry
    lax.fori_loop(0, 8, fill_ones, 0)

    for t in range(ROWS_PER_TILE // CHUNK):
        pltpu.sync_copy(rows_v, acc_sh.at[pl.ds(s * ROWS_PER_TILE + t * CHUNK, CHUNK)])

    pltpu.sync_copy(zrow_v, deg_sh.at[pl.ds(s * DEG_STRIDE, DEG_STRIDE)])

    pltpu.sync_copy(src_hbm.at[wid], src_v)
    pltpu.sync_copy(dst_hbm.at[wid], dst_v)

    plsc.subcore_barrier()

    def body(j, carry):
        pltpu.async_copy(x_hbm.at[src_v.at[j]], rows_v, gsem).wait()
        pltpu.sync_copy(rows_v, acc_sh.at[dst_v.at[j]], add=True)
        pltpu.sync_copy(ones_v.at[pl.ds(0, CHUNK)], deg_sh.at[dst_v.at[j]], add=True)
        return carry
    lax.fori_loop(0, CHUNKS, body, 0)

    plsc.subcore_barrier()

    pltpu.sync_copy(acc_sh.at[pl.ds(s * ROWS_PER_TILE, ROWS_PER_TILE)],
                    acc_out.at[c, s])
    pltpu.sync_copy(deg_sh.at[pl.ds(s * DEG_STRIDE, DEG_STRIDE)],
                    deg_out.at[c, s])


_sc_aggregate = functools.partial(
    pl.kernel,
    mesh=plsc.VectorSubcoreMesh(core_axis_name="c", subcore_axis_name="s"),
    out_type=[
        jax.ShapeDtypeStruct((NC, NS, ROWS_PER_TILE, D), jnp.float32),
        jax.ShapeDtypeStruct((NC, NS, DEG_STRIDE), jnp.float32),
    ],
    scratch_types=[
        pltpu.VMEM((CHUNKS, CHUNK), jnp.int32),
        pltpu.VMEM((CHUNKS, CHUNK), jnp.int32),
        pltpu.VMEM((CHUNK, D), jnp.float32),
        pltpu.VMEM((D,), jnp.float32),
        pltpu.VMEM((DEG_STRIDE,), jnp.float32),
        pltpu.VMEM_SHARED((N_NODES, D), jnp.float32),
        pltpu.VMEM_SHARED((N_PAD,), jnp.float32),
        pltpu.SemaphoreType.DMA,
    ],
)(_sc_aggregate_body)


BM = 2000


def _finish_body(acc_ref, deg_ref, w_ref, b_ref, o_ref):
    a = acc_ref[0] + acc_ref[1]
    dp = deg_ref[...]
    d = jnp.maximum(dp[:, 0] + dp[:, 1], 1.0)
    y = jnp.dot(a, w_ref[...], preferred_element_type=jnp.float32)
    o_ref[...] = y / d[:, None] + b_ref[...]


def _finish(acc, deg, W, b2):
    return pl.pallas_call(
        _finish_body,
        grid=(N_NODES // BM,),
        in_specs=[
            pl.BlockSpec((NC, BM, D), lambda i: (0, i, 0)),
            pl.BlockSpec((BM, NC), lambda i: (i, 0)),
            pl.BlockSpec((D, D), lambda i: (0, 0)),
            pl.BlockSpec((1, D), lambda i: (0, 0)),
        ],
        out_specs=pl.BlockSpec((BM, D), lambda i: (i, 0)),
        out_shape=jax.ShapeDtypeStruct((N_NODES, D), jnp.float32),
    )(acc, deg, W, b2)


def kernel(x, edge_index, W, b):
    dst = edge_index[0].reshape(NW, CHUNKS, CHUNK)
    src = edge_index[1].reshape(NW, CHUNKS, CHUNK)
    acc, deg = _sc_aggregate(x, src, dst)
    acc = acc.reshape(NC, N_NODES, D)
    deg = deg.reshape(NC, N_PAD)[:, :N_NODES].T
    return _finish(acc, deg, W, b.reshape(1, D))
